# baseline (speedup 1.0000x reference)
import functools

import jax
import jax.numpy as jnp
from jax import lax
from jax.experimental import pallas as pl
from jax.experimental.pallas import tpu as pltpu
from jax.experimental.pallas import tpu_sc as plsc

_SC_CORES = 2
_SC_SUBCORES = 16
_SC_WORKERS = _SC_CORES * _SC_SUBCORES
_CHUNK = 128


def _onehot2(ids2d, k):
    a, b2 = ids2d.shape
    ids3 = jax.lax.broadcast_in_dim(ids2d, (a, b2, k), (0, 1))
    iota = jax.lax.broadcasted_iota(jnp.int32, (a, b2, k), 2)
    return (ids3 == iota).astype(jnp.float32).reshape(a * b2, k)


def _m_sc_body(nch, idx_hbm, table_hbm, out_hbm, table_v, idx_v, rows_v,
               ssem):
    ch = idx_v.shape[1]
    cm = table_v.shape[1]
    ng = cm // 16
    wid = lax.axis_index("s") * _SC_CORES + lax.axis_index("c")
    pltpu.sync_copy(table_hbm, table_v)
    pltpu.sync_copy(idx_hbm.at[wid], idx_v)
    base = wid * (nch * ch)

    def scatter(c, buf):
        return pltpu.make_async_copy(
            rows_v.at[buf], out_hbm.at[pl.ds(base + c * ch, ch)], ssem)

    def chunk_body(c, _):
        buf = lax.rem(c, 2)

        @pl.when(c >= 2)
        def _wait_prev():
            scatter(c - 2, buf).wait()

        def group_body(q, _):
            idx16 = idx_v[c, pl.ds(q * 16, 16)]
            for r16 in range(16):
                s = idx16[r16]
                r = q * 16 + r16
                for g in range(ng):
                    sl = pl.ds(g * 16, 16)
                    rows_v[buf, r, sl] = table_v[s, sl]
            return 0

        lax.fori_loop(0, ch // 16, group_body, 0)
        scatter(c, buf).start()
        return 0

    lax.fori_loop(0, nch, chunk_body, 0)
    scatter(nch - 1, 0).wait()
    scatter(nch - 2, 0).wait()


def _z_body(afull_ref, ablk_ref, semb_ref, remb_ref, w_ref, b_ref, out_ref):
    l = afull_ref.shape[1]
    ib = ablk_ref.shape[2]
    na, ch = semb_ref.shape
    nr = remb_ref.shape[0]

    semb = semb_ref[...]
    s_full = jnp.dot(_onehot2(afull_ref[...], na), semb,
                     preferred_element_type=jnp.float32)
    s_blk = jnp.dot(_onehot2(ablk_ref[...].reshape(1, ib), na), semb,
                    preferred_element_type=jnp.float32)

    ptab = jnp.dot(remb_ref[...], w_ref[...],
                   preferred_element_type=jnp.float32) + b_ref[...]

    i0 = pl.program_id(0) * ib
    ivec = i0 + jax.lax.broadcasted_iota(jnp.int32, (ib, l), 0)
    jvec = jax.lax.broadcasted_iota(jnp.int32, (ib, l), 1)
    rel = jnp.clip(ivec - jvec, -32, 32) + 32
    pt = jnp.dot(_onehot2(rel, nr), ptab,
                 preferred_element_type=jnp.float32)

    zeros_i = jnp.zeros((ib, ch), jnp.float32)
    zeros_j = jnp.zeros((l, ch), jnp.float32)
    si = jnp.concatenate([s_blk, zeros_i], axis=-1)
    sj = jnp.concatenate([zeros_j, s_full], axis=-1)
    z = pt.reshape(ib, l, 2 * ch) + si[:, None, :] + sj[None, :, :]
    out_ref[...] = z.reshape(1, ib, l, 2 * ch)


def kernel(aatype, msa, msa_emb, seq_emb, relpos_emb, relpos_W, relpos_b):
    b, n, l = msa.shape
    k, cm = msa_emb.shape
    ch = seq_emb.shape[1]
    cz = 2 * ch

    aat2 = aatype.reshape(b, l).astype(jnp.int32)

    total = b * n * l
    nch = total // (_SC_WORKERS * _CHUNK)
    msa3 = msa.reshape(_SC_WORKERS, nch, _CHUNK).astype(jnp.int32)
    m_flat = pl.kernel(
        functools.partial(_m_sc_body, nch),
        out_type=jax.ShapeDtypeStruct((total, cm), jnp.float32),
        mesh=plsc.VectorSubcoreMesh(core_axis_name="c", subcore_axis_name="s"),
        scratch_types=[
            pltpu.VMEM((k, cm), jnp.float32),
            pltpu.VMEM((nch, _CHUNK), jnp.int32),
            pltpu.VMEM((2, _CHUNK, cm), jnp.float32),
            pltpu.SemaphoreType.DMA,
        ],
    )(msa3, msa_emb)
    m = m_flat.reshape(b, n, l, cm)

    ib = 32
    z = pl.pallas_call(
        _z_body,
        grid=(l // ib,),
        in_specs=[
            pl.BlockSpec((1, l), lambda i: (0, 0)),
            pl.BlockSpec((1, 1, ib), lambda i: (i, 0, 0)),
            pl.BlockSpec((k, ch), lambda i: (0, 0)),
            pl.BlockSpec((65, ch), lambda i: (0, 0)),
            pl.BlockSpec((ch, cz), lambda i: (0, 0)),
            pl.BlockSpec((1, cz), lambda i: (0, 0)),
        ],
        out_specs=pl.BlockSpec((1, ib, l, cz), lambda i: (0, i, 0, 0)),
        out_shape=jax.ShapeDtypeStruct((1, l, l, cz), jnp.float32),
    )(aat2, aat2.reshape(b * l // ib, 1, ib), seq_emb, relpos_emb, relpos_W,
      relpos_b.reshape(1, cz))
    z = jnp.broadcast_to(z, (b, l, l, cz))

    return (m, z)


# speedup vs baseline: 2.3729x; 2.3729x over previous
import functools

import jax
import jax.numpy as jnp
from jax import lax
from jax.experimental import pallas as pl
from jax.experimental.pallas import tpu as pltpu
from jax.experimental.pallas import tpu_sc as plsc

_SC_CORES = 2
_SC_SUBCORES = 16
_SC_WORKERS = _SC_CORES * _SC_SUBCORES
_CHUNK = 128


def _onehot2(ids2d, k):
    a, b2 = ids2d.shape
    ids3 = jax.lax.broadcast_in_dim(ids2d, (a, b2, k), (0, 1))
    iota = jax.lax.broadcasted_iota(jnp.int32, (a, b2, k), 2)
    return (ids3 == iota).astype(jnp.float32).reshape(a * b2, k)


def _m_sc_body(nch, idx_hbm, table_hbm, out_hbm, table_v, idx_v, rows_v,
               ssem):
    ch = idx_v.shape[1]
    cm = table_v.shape[1]
    ng = cm // 16
    wid = lax.axis_index("s") * _SC_CORES + lax.axis_index("c")
    pltpu.sync_copy(table_hbm, table_v)
    pltpu.sync_copy(idx_hbm.at[wid], idx_v)
    base = wid * (nch * ch)

    def scatter(c, buf):
        return pltpu.make_async_copy(
            rows_v.at[buf], out_hbm.at[pl.ds(base + c * ch, ch)], ssem)

    def chunk_body(c, _):
        buf = lax.rem(c, 2)

        @pl.when(c >= 2)
        def _wait_prev():
            scatter(c - 2, buf).wait()

        def group_body(q, _):
            idx16 = idx_v[c, pl.ds(q * 16, 16)]
            for r16 in range(16):
                s = idx16[r16]
                r = q * 16 + r16
                vals = [table_v[s, pl.ds(g * 16, 16)] for g in range(ng)]
                for g in range(ng):
                    rows_v[buf, r, pl.ds(g * 16, 16)] = vals[g]
            return 0

        lax.fori_loop(0, ch // 16, group_body, 0)
        scatter(c, buf).start()
        return 0

    lax.fori_loop(0, nch, chunk_body, 0)
    scatter(nch - 1, 0).wait()
    scatter(nch - 2, 0).wait()


def _z_body(afull_ref, ablk_ref, semb_ref, remb_ref, w_ref, b_ref, out_ref):
    l = afull_ref.shape[1]
    ib = ablk_ref.shape[2]
    na, ch = semb_ref.shape
    nr = remb_ref.shape[0]

    semb = semb_ref[...]
    s_full = jnp.dot(_onehot2(afull_ref[...], na), semb,
                     preferred_element_type=jnp.float32)
    s_blk = jnp.dot(_onehot2(ablk_ref[...].reshape(1, ib), na), semb,
                    preferred_element_type=jnp.float32)

    ptab = jnp.dot(remb_ref[...], w_ref[...],
                   preferred_element_type=jnp.float32) + b_ref[...]

    i0 = pl.program_id(0) * ib
    ivec = i0 + jax.lax.broadcasted_iota(jnp.int32, (ib, l), 0)
    jvec = jax.lax.broadcasted_iota(jnp.int32, (ib, l), 1)
    rel = jnp.clip(ivec - jvec, -32, 32) + 32
    pt = jnp.dot(_onehot2(rel, nr), ptab,
                 preferred_element_type=jnp.float32)

    zeros_i = jnp.zeros((ib, ch), jnp.float32)
    zeros_j = jnp.zeros((l, ch), jnp.float32)
    si = jnp.concatenate([s_blk, zeros_i], axis=-1)
    sj = jnp.concatenate([zeros_j, s_full], axis=-1)
    z = pt.reshape(ib, l, 2 * ch) + si[:, None, :] + sj[None, :, :]
    out_ref[...] = z.reshape(1, ib, l, 2 * ch)


def kernel(aatype, msa, msa_emb, seq_emb, relpos_emb, relpos_W, relpos_b):
    b, n, l = msa.shape
    k, cm = msa_emb.shape
    ch = seq_emb.shape[1]
    cz = 2 * ch

    aat2 = aatype.reshape(b, l).astype(jnp.int32)

    total = b * n * l
    nch = total // (_SC_WORKERS * _CHUNK)
    msa3 = msa.reshape(_SC_WORKERS, nch, _CHUNK).astype(jnp.int32)
    m_flat = pl.kernel(
        functools.partial(_m_sc_body, nch),
        out_type=jax.ShapeDtypeStruct((total, cm), jnp.float32),
        mesh=plsc.VectorSubcoreMesh(core_axis_name="c", subcore_axis_name="s"),
        scratch_types=[
            pltpu.VMEM((k, cm), jnp.float32),
            pltpu.VMEM((nch, _CHUNK), jnp.int32),
            pltpu.VMEM((2, _CHUNK, cm), jnp.float32),
            pltpu.SemaphoreType.DMA,
        ],
    )(msa3, msa_emb)
    m = m_flat.reshape(b, n, l, cm)

    ib = 32
    z = pl.pallas_call(
        _z_body,
        grid=(l // ib,),
        in_specs=[
            pl.BlockSpec((1, l), lambda i: (0, 0)),
            pl.BlockSpec((1, 1, ib), lambda i: (i, 0, 0)),
            pl.BlockSpec((k, ch), lambda i: (0, 0)),
            pl.BlockSpec((65, ch), lambda i: (0, 0)),
            pl.BlockSpec((ch, cz), lambda i: (0, 0)),
            pl.BlockSpec((1, cz), lambda i: (0, 0)),
        ],
        out_specs=pl.BlockSpec((1, ib, l, cz), lambda i: (0, i, 0, 0)),
        out_shape=jax.ShapeDtypeStruct((1, l, l, cz), jnp.float32),
    )(aat2, aat2.reshape(b * l // ib, 1, ib), seq_emb, relpos_emb, relpos_W,
      relpos_b.reshape(1, cz))
    z = jnp.broadcast_to(z, (b, l, l, cz))

    return (m, z)


# speedup vs baseline: 2.5599x; 1.0788x over previous
import functools

import jax
import jax.numpy as jnp
from jax import lax
from jax.experimental import pallas as pl
from jax.experimental.pallas import tpu as pltpu
from jax.experimental.pallas import tpu_sc as plsc

_SC_CORES = 2
_SC_SUBCORES = 16
_SC_WORKERS = _SC_CORES * _SC_SUBCORES
_CHUNK = 128


def _onehot2(ids2d, k):
    a, b2 = ids2d.shape
    ids3 = jax.lax.broadcast_in_dim(ids2d, (a, b2, k), (0, 1))
    iota = jax.lax.broadcasted_iota(jnp.int32, (a, b2, k), 2)
    return (ids3 == iota).astype(jnp.float32).reshape(a * b2, k)


def _m_sc_body(nch, idx_hbm, table_hbm, out_hbm, table_v, idx_v, ssem):
    ch = idx_v.shape[1]
    wid = lax.axis_index("s") * _SC_CORES + lax.axis_index("c")
    pltpu.sync_copy(table_hbm, table_v)
    pltpu.sync_copy(idx_hbm.at[wid], idx_v)
    base = wid * (nch * ch)
    ngroups = (nch * ch) // 16

    def row_copy(s, row):
        return pltpu.make_async_copy(
            table_v.at[pl.ds(s, 1)], out_hbm.at[pl.ds(row, 1)], ssem)

    def group_body(q, _):
        c = q // (ch // 16)
        q16 = lax.rem(q, ch // 16) * 16
        idx16 = idx_v[c, pl.ds(q16, 16)]
        for r16 in range(16):
            s = idx16[r16]
            row_copy(s, base + c * ch + q16 + r16).start()

        @pl.when(q >= 2)
        def _drain_lagged():
            for _ in range(16):
                row_copy(0, base).wait()
        return 0

    lax.fori_loop(0, ngroups, group_body, 0)
    for _ in range(32):
        row_copy(0, base).wait()


def _z_body(afull_ref, ablk_ref, semb_ref, remb_ref, w_ref, b_ref, out_ref):
    l = afull_ref.shape[1]
    ib = ablk_ref.shape[2]
    na, ch = semb_ref.shape
    nr = remb_ref.shape[0]

    semb = semb_ref[...]
    s_full = jnp.dot(_onehot2(afull_ref[...], na), semb,
                     preferred_element_type=jnp.float32)
    s_blk = jnp.dot(_onehot2(ablk_ref[...].reshape(1, ib), na), semb,
                    preferred_element_type=jnp.float32)

    ptab = jnp.dot(remb_ref[...], w_ref[...],
                   preferred_element_type=jnp.float32) + b_ref[...]

    i0 = pl.program_id(0) * ib
    ivec = i0 + jax.lax.broadcasted_iota(jnp.int32, (ib, l), 0)
    jvec = jax.lax.broadcasted_iota(jnp.int32, (ib, l), 1)
    rel = jnp.clip(ivec - jvec, -32, 32) + 32
    pt = jnp.dot(_onehot2(rel, nr), ptab,
                 preferred_element_type=jnp.float32)

    zeros_i = jnp.zeros((ib, ch), jnp.float32)
    zeros_j = jnp.zeros((l, ch), jnp.float32)
    si = jnp.concatenate([s_blk, zeros_i], axis=-1)
    sj = jnp.concatenate([zeros_j, s_full], axis=-1)
    z = pt.reshape(ib, l, 2 * ch) + si[:, None, :] + sj[None, :, :]
    out_ref[...] = z.reshape(1, ib, l, 2 * ch)


def kernel(aatype, msa, msa_emb, seq_emb, relpos_emb, relpos_W, relpos_b):
    b, n, l = msa.shape
    k, cm = msa_emb.shape
    ch = seq_emb.shape[1]
    cz = 2 * ch

    aat2 = aatype.reshape(b, l).astype(jnp.int32)

    total = b * n * l
    nch = total // (_SC_WORKERS * _CHUNK)
    msa3 = msa.reshape(_SC_WORKERS, nch, _CHUNK).astype(jnp.int32)
    m_flat = pl.kernel(
        functools.partial(_m_sc_body, nch),
        out_type=jax.ShapeDtypeStruct((total, cm), jnp.float32),
        mesh=plsc.VectorSubcoreMesh(core_axis_name="c", subcore_axis_name="s"),
        scratch_types=[
            pltpu.VMEM((k, cm), jnp.float32),
            pltpu.VMEM((nch, _CHUNK), jnp.int32),
            pltpu.SemaphoreType.DMA,
        ],
    )(msa3, msa_emb)
    m = m_flat.reshape(b, n, l, cm)

    ib = 32
    z = pl.pallas_call(
        _z_body,
        grid=(l // ib,),
        in_specs=[
            pl.BlockSpec((1, l), lambda i: (0, 0)),
            pl.BlockSpec((1, 1, ib), lambda i: (i, 0, 0)),
            pl.BlockSpec((k, ch), lambda i: (0, 0)),
            pl.BlockSpec((65, ch), lambda i: (0, 0)),
            pl.BlockSpec((ch, cz), lambda i: (0, 0)),
            pl.BlockSpec((1, cz), lambda i: (0, 0)),
        ],
        out_specs=pl.BlockSpec((1, ib, l, cz), lambda i: (0, i, 0, 0)),
        out_shape=jax.ShapeDtypeStruct((1, l, l, cz), jnp.float32),
    )(aat2, aat2.reshape(b * l // ib, 1, ib), seq_emb, relpos_emb, relpos_W,
      relpos_b.reshape(1, cz))
    z = jnp.broadcast_to(z, (b, l, l, cz))

    return (m, z)
